# Initial kernel scaffold; baseline (speedup 1.0000x reference)
import jax
import jax.numpy as jnp
from jax.experimental import pallas as pl


def kernel(x, edge_index, W_gcn, b_gcn, alpha, W_d1, b_d1, W_d2, b_d2):
    raise NotImplementedError("write your pallas kernel here")


# speedup vs baseline: 35.3640x; 35.3640x over previous
import functools
import math

import jax
import jax.numpy as jnp
import numpy as np
from jax import lax
from jax.experimental import pallas as pl
from jax.experimental.pallas import tpu as pltpu
from jax.experimental.pallas import tpu_sc as plsc

N = 10000
E = 320000
D = 128
NS = 16
NC = 2
WIN = E // NS
NPAD = 10240
ROWS_PER_TILE = NPAD // NS
CHUNK = 64


def _tf2x32(k1, k2, x0, x1):
    rot = [np.array([13, 15, 26, 6], np.uint32),
           np.array([17, 29, 16, 24], np.uint32)]

    def rl(x, d):
        return ((x << d) | (x >> np.uint32(32 - d))).astype(np.uint32)

    ks = [k1, k2, (k1 ^ k2 ^ np.uint32(0x1BD11BDA)).astype(np.uint32)]
    x = [(x0 + ks[0]).astype(np.uint32), (x1 + ks[1]).astype(np.uint32)]
    for i in range(5):
        for r in rot[i % 2]:
            x[0] = (x[0] + x[1]).astype(np.uint32)
            x[1] = x[0] ^ rl(x[1], r)
        x[0] = (x[0] + ks[(i + 1) % 3]).astype(np.uint32)
        x[1] = (x[1] + ks[(i + 2) % 3] + np.uint32(i + 1)).astype(np.uint32)
    return x


def _np_split(key, num):
    hi = np.zeros(num, np.uint32)
    lo = np.arange(num, dtype=np.uint32)
    b1, b2 = _tf2x32(key[0], key[1], hi, lo)
    return np.stack([b1, b2], axis=1)


def _np_uniform(key, n):
    hi = np.zeros(n, np.uint32)
    lo = np.arange(n, dtype=np.uint32)
    b1, b2 = _tf2x32(key[0], key[1], hi, lo)
    fb = ((b1 ^ b2) >> np.uint32(9)) | np.uint32(0x3F800000)
    return fb.view(np.float32) - np.float32(1.0)


def _build_static_masks():
    ks = _np_split(np.array([0, 42], np.uint32), 4)
    keep1 = _np_uniform(ks[0], E) >= np.float32(0.8)
    keep2 = _np_uniform(ks[1], E) >= np.float32(0.7)
    fm1 = (_np_uniform(ks[2], D) >= np.float32(0.4)).astype(np.float32)
    fm2 = (_np_uniform(ks[3], D) >= np.float32(0.3)).astype(np.float32)
    return keep1, keep2, fm1, fm2


def _build_tables():
    keep1, keep2, fm1, fm2 = _build_static_masks()
    counts = np.zeros((NC, NS), np.int32)
    offs = [[None] * NS for _ in range(NC)]
    for v, keep in enumerate((keep1, keep2)):
        for t in range(NS):
            o = (np.nonzero(keep[t * WIN:(t + 1) * WIN])[0] + t * WIN).astype(
                np.int32)
            counts[v, t] = o.size
            offs[v][t] = o
    nchunk = int(math.ceil(counts.max() / CHUNK))
    eid = np.tile(np.arange(nchunk * CHUNK, dtype=np.int32) * 51 % E,
                  (NC, NS, 1))
    for v in range(NC):
        for t in range(NS):
            eid[v, t, : counts[v, t]] = offs[v][t]
    return (eid.reshape(NC, NS, nchunk * CHUNK), counts, fm1, fm2, nchunk)


_EID, _COUNTS, _FM1, _FM2, NCHUNK = _build_tables()


def _sc1_body(src_ref, dst_ref, eid_ref, cnt_ref,
              deg_ref, srcact_ref, dstact_ref,
              eid_v, src_f, dgat, dstact, cnt_v, ones_v, degbuf,
              deg_sp, sem):
    c = lax.axis_index("c")
    s = lax.axis_index("s")
    tile_r0 = s * ROWS_PER_TILE
    view_r0 = c * NPAD

    pltpu.sync_copy(cnt_ref.at[c], cnt_v)
    pltpu.sync_copy(eid_ref.at[c, s], eid_v)
    lanes = lax.iota(jnp.int32, 16)
    cnt = jnp.sum(jnp.where(lanes == s, cnt_v[...], jnp.int32(0)))
    nj = lax.div(cnt + jnp.int32(CHUNK - 1), jnp.int32(CHUNK))

    pltpu.async_copy(src_ref.at[eid_v], src_f, sem).wait()
    for k in range(CHUNK // 16):
        degbuf[pl.ds(k * 16, 16)] = jnp.zeros((16,), jnp.float32)
        ones_v[pl.ds(k * 16, 16)] = jnp.ones((16,), jnp.float32)
    for q in range(ROWS_PER_TILE // CHUNK):
        pltpu.sync_copy(degbuf, deg_sp.at[pl.ds(tile_r0 + q * CHUNK, CHUNK)])
    plsc.subcore_barrier()

    @pl.loop(0, nj)
    def _phase_a(j):
        eid_j = eid_v.at[pl.ds(j * CHUNK, CHUNK)]
        pltpu.async_copy(dst_ref.at[eid_j], dgat, sem).wait()
        for k in range(CHUNK // 16):
            sv = src_f[pl.ds(j * CHUNK + k * 16, 16)]
            dv = dgat[pl.ds(k * 16, 16)]
            pos = j * CHUNK + k * 16 + lanes
            live = pos < cnt
            dump = jnp.int32(N) + (pos & jnp.int32(127))
            spread = (pos * jnp.int32(37)) & jnp.int32(8191)
            src_f[pl.ds(j * CHUNK + k * 16, 16)] = (
                jnp.where(live, sv, spread) + view_r0)
            dstact[j, pl.ds(k * 16, 16)] = jnp.where(live, dv, dump)
        pltpu.sync_copy(ones_v, deg_sp.at[dstact.at[j]], add=True)

    plsc.subcore_barrier()

    pltpu.sync_copy(src_f, srcact_ref.at[c, s])
    pltpu.sync_copy(dstact, dstact_ref.at[c, s])
    pltpu.sync_copy(deg_sp.at[pl.ds(tile_r0, ROWS_PER_TILE)],
                    deg_ref.at[pl.ds(view_r0 + tile_r0, ROWS_PER_TILE)])


def _sc1(src, dst, eid, counts):
    mesh = plsc.VectorSubcoreMesh(core_axis_name="c", subcore_axis_name="s")
    kern = pl.kernel(
        _sc1_body,
        out_type=[
            jax.ShapeDtypeStruct((NC * NPAD,), jnp.float32),
            jax.ShapeDtypeStruct((NC, NS, NCHUNK * CHUNK), jnp.int32),
            jax.ShapeDtypeStruct((NC, NS, NCHUNK, CHUNK), jnp.int32),
        ],
        mesh=mesh,
        scratch_types=[
            pltpu.VMEM((NCHUNK * CHUNK,), jnp.int32),
            pltpu.VMEM((NCHUNK * CHUNK,), jnp.int32),
            pltpu.VMEM((CHUNK,), jnp.int32),
            pltpu.VMEM((NCHUNK, CHUNK), jnp.int32),
            pltpu.VMEM((16,), jnp.int32),
            pltpu.VMEM((CHUNK,), jnp.float32),
            pltpu.VMEM((CHUNK,), jnp.float32),
            pltpu.MemorySpace.VMEM_SHARED((NPAD,), jnp.float32),
            pltpu.SemaphoreType.DMA,
        ],
        compiler_params=pltpu.CompilerParams(needs_layout_passes=False),
    )
    return kern(src, dst, eid, counts)


def _sc2_body(u_ref, srcact_ref, dstact_ref, cnt_ref,
              agg_ref,
              src_f, dstact, cnt_v, rowbuf, zbuf,
              agg_sp, sem):
    c = lax.axis_index("c")
    s = lax.axis_index("s")
    tile_r0 = s * ROWS_PER_TILE
    view_r0 = c * NPAD

    pltpu.sync_copy(cnt_ref.at[c], cnt_v)
    pltpu.sync_copy(srcact_ref.at[c, s], src_f)
    pltpu.sync_copy(dstact_ref.at[c, s], dstact)
    lanes = lax.iota(jnp.int32, 16)
    cnt = jnp.sum(jnp.where(lanes == s, cnt_v[...], jnp.int32(0)))
    nj = lax.div(cnt + jnp.int32(CHUNK - 1), jnp.int32(CHUNK))

    @pl.loop(0, CHUNK)
    def _zero_zbuf(r):
        for k in range(D // 16):
            zbuf[r, pl.ds(k * 16, 16)] = jnp.zeros((16,), jnp.float32)

    for q in range(ROWS_PER_TILE // CHUNK):
        pltpu.sync_copy(zbuf, agg_sp.at[pl.ds(tile_r0 + q * CHUNK, CHUNK)])
    plsc.subcore_barrier()

    @pl.loop(0, nj)
    def _phase_c(j):
        idx = src_f.at[pl.ds(j * CHUNK, CHUNK)]
        pltpu.async_copy(u_ref.at[idx], rowbuf, sem).wait()
        pltpu.sync_copy(rowbuf, agg_sp.at[dstact.at[j]], add=True)

    plsc.subcore_barrier()

    pltpu.sync_copy(agg_sp.at[pl.ds(tile_r0, ROWS_PER_TILE)],
                    agg_ref.at[pl.ds(view_r0 + tile_r0, ROWS_PER_TILE)])


def _sc2(u, srcact, dstact, counts):
    mesh = plsc.VectorSubcoreMesh(core_axis_name="c", subcore_axis_name="s")
    kern = pl.kernel(
        _sc2_body,
        out_type=[
            jax.ShapeDtypeStruct((NC * NPAD, D), jnp.float32),
        ],
        mesh=mesh,
        scratch_types=[
            pltpu.VMEM((NCHUNK * CHUNK,), jnp.int32),
            pltpu.VMEM((NCHUNK, CHUNK), jnp.int32),
            pltpu.VMEM((16,), jnp.int32),
            pltpu.VMEM((CHUNK, D), jnp.float32),
            pltpu.VMEM((CHUNK, D), jnp.float32),
            pltpu.MemorySpace.VMEM_SHARED((NPAD, D), jnp.float32),
            pltpu.SemaphoreType.DMA,
        ],
        compiler_params=pltpu.CompilerParams(needs_layout_passes=False),
    )
    (agg,) = kern(u, srcact, dstact, counts)
    return agg


def _tc1_body(deg_ref, x_ref, u_ref, dinv_ref):
    dinv = lax.rsqrt(deg_ref[...] + jnp.float32(1.0))
    dinv_ref[...] = dinv
    u_ref[...] = x_ref[...] * dinv


def _tc1(deg2, x_pad):
    blk = 1024
    nb = NPAD // blk
    return pl.pallas_call(
        _tc1_body,
        grid=(NC, nb),
        in_specs=[
            pl.BlockSpec((blk, 1), lambda c, j: (c * nb + j, 0)),
            pl.BlockSpec((blk, D), lambda c, j: (j, 0)),
        ],
        out_specs=[
            pl.BlockSpec((blk, D), lambda c, j: (c * nb + j, 0)),
            pl.BlockSpec((blk, 1), lambda c, j: (c * nb + j, 0)),
        ],
        out_shape=[
            jax.ShapeDtypeStruct((NC * NPAD, D), jnp.float32),
            jax.ShapeDtypeStruct((NC * NPAD, 1), jnp.float32),
        ],
    )(deg2.reshape(NC * NPAD, 1), x_pad)


def _tc2_body(alpha_ref,
              a1_ref, u1_ref, d1_ref, a2_ref, u2_ref, d2_ref,
              w1_ref, w2_ref, bg_ref, wd1_ref, bd1_ref, wd2_ref, bd2_ref,
              o1_ref, o2_ref):
    alpha = alpha_ref[0]
    dot = functools.partial(
        lax.dot_general,
        dimension_numbers=(((1,), (0,)), ((), ())),
        precision=lax.Precision.HIGHEST,
        preferred_element_type=jnp.float32,
    )
    for a_ref, uu_ref, dd_ref, w_ref, o_ref in (
            (a1_ref, u1_ref, d1_ref, w1_ref, o1_ref),
            (a2_ref, u2_ref, d2_ref, w2_ref, o2_ref)):
        zin = (a_ref[...] + uu_ref[...]) * dd_ref[...]
        z = dot(zin, w_ref[...]) + bg_ref[...]
        z = jnp.where(z > 0, z, alpha * z)
        e = dot(z, wd1_ref[...]) + bd1_ref[...]
        e = jnp.where(e > 0, e, jnp.exp(e) - jnp.float32(1.0))
        o_ref[...] = dot(e, wd2_ref[...]) + bd2_ref[...]


def _tc2(agg, u, dinv, w1e, w2e, b_gcn, alpha, w_d1, b_d1, w_d2, b_d2):
    blk = 512
    nb = NPAD // blk
    rows1 = pl.BlockSpec((blk, D), lambda i: (i, 0))
    rows2 = pl.BlockSpec((blk, D), lambda i: (nb + i, 0))
    dcol1 = pl.BlockSpec((blk, 1), lambda i: (i, 0))
    dcol2 = pl.BlockSpec((blk, 1), lambda i: (nb + i, 0))
    full = pl.BlockSpec((D, D), lambda i: (0, 0))
    vec = pl.BlockSpec((1, D), lambda i: (0, 0))
    return pl.pallas_call(
        _tc2_body,
        grid=(nb,),
        in_specs=[
            pl.BlockSpec(memory_space=pltpu.SMEM),
            rows1, rows1, dcol1, rows2, rows2, dcol2,
            full, full, vec, full, vec, full, vec,
        ],
        out_specs=[rows1, rows1],
        out_shape=[
            jax.ShapeDtypeStruct((NPAD, D), jnp.float32),
            jax.ShapeDtypeStruct((NPAD, D), jnp.float32),
        ],
    )(alpha.reshape(1), agg, u, dinv, agg, u, dinv,
      w1e, w2e, b_gcn.reshape(1, D),
      w_d1, b_d1.reshape(1, D), w_d2, b_d2.reshape(1, D))


def kernel(x, edge_index, W_gcn, b_gcn, alpha, W_d1, b_d1, W_d2, b_d2):
    x_pad = jnp.pad(x, ((0, NPAD - N), (0, 0)))
    src = edge_index[0].astype(jnp.int32)
    dst = edge_index[1].astype(jnp.int32)
    eid = jnp.asarray(_EID)
    counts = jnp.asarray(_COUNTS)

    deg2, srcact, dstact = _sc1(src, dst, eid, counts)
    u, dinv = _tc1(deg2, x_pad)
    agg = _sc2(u, srcact, dstact, counts)

    w1e = W_gcn * jnp.asarray(_FM1).reshape(D, 1)
    w2e = W_gcn * jnp.asarray(_FM2).reshape(D, 1)

    h1, h2 = _tc2(agg, u, dinv, w1e, w2e, b_gcn, alpha,
                  W_d1, b_d1, W_d2, b_d2)
    return (h1[:N], h2[:N])


# speedup vs baseline: 49.8200x; 1.4088x over previous
import functools
import math

import jax
import jax.numpy as jnp
import numpy as np
from jax import lax
from jax.experimental import pallas as pl
from jax.experimental.pallas import tpu as pltpu
from jax.experimental.pallas import tpu_sc as plsc

N = 10000
E = 320000
D = 128
NS = 16
NC = 2
WIN = E // NS
NPAD = 10240
ROWS_PER_TILE = NPAD // NS
CHUNK = 128


def _tf2x32(k1, k2, x0, x1):
    rot = [np.array([13, 15, 26, 6], np.uint32),
           np.array([17, 29, 16, 24], np.uint32)]

    def rl(x, d):
        return ((x << d) | (x >> np.uint32(32 - d))).astype(np.uint32)

    ks = [k1, k2, (k1 ^ k2 ^ np.uint32(0x1BD11BDA)).astype(np.uint32)]
    x = [(x0 + ks[0]).astype(np.uint32), (x1 + ks[1]).astype(np.uint32)]
    for i in range(5):
        for r in rot[i % 2]:
            x[0] = (x[0] + x[1]).astype(np.uint32)
            x[1] = x[0] ^ rl(x[1], r)
        x[0] = (x[0] + ks[(i + 1) % 3]).astype(np.uint32)
        x[1] = (x[1] + ks[(i + 2) % 3] + np.uint32(i + 1)).astype(np.uint32)
    return x


def _np_split(key, num):
    hi = np.zeros(num, np.uint32)
    lo = np.arange(num, dtype=np.uint32)
    b1, b2 = _tf2x32(key[0], key[1], hi, lo)
    return np.stack([b1, b2], axis=1)


def _np_uniform(key, n):
    hi = np.zeros(n, np.uint32)
    lo = np.arange(n, dtype=np.uint32)
    b1, b2 = _tf2x32(key[0], key[1], hi, lo)
    fb = ((b1 ^ b2) >> np.uint32(9)) | np.uint32(0x3F800000)
    return fb.view(np.float32) - np.float32(1.0)


def _build_static_masks():
    ks = _np_split(np.array([0, 42], np.uint32), 4)
    keep1 = _np_uniform(ks[0], E) >= np.float32(0.8)
    keep2 = _np_uniform(ks[1], E) >= np.float32(0.7)
    fm1 = (_np_uniform(ks[2], D) >= np.float32(0.4)).astype(np.float32)
    fm2 = (_np_uniform(ks[3], D) >= np.float32(0.3)).astype(np.float32)
    return keep1, keep2, fm1, fm2


def _build_tables():
    keep1, keep2, fm1, fm2 = _build_static_masks()
    counts = np.zeros((NC, NS), np.int32)
    offs = [[None] * NS for _ in range(NC)]
    for v, keep in enumerate((keep1, keep2)):
        for t in range(NS):
            o = (np.nonzero(keep[t * WIN:(t + 1) * WIN])[0] + t * WIN).astype(
                np.int32)
            counts[v, t] = o.size
            offs[v][t] = o
    nchunk = int(math.ceil(counts.max() / CHUNK))
    eid = np.tile(np.arange(nchunk * CHUNK, dtype=np.int32) * 51 % E,
                  (NC, NS, 1))
    for v in range(NC):
        for t in range(NS):
            eid[v, t, : counts[v, t]] = offs[v][t]
    return (eid.reshape(NC, NS, nchunk * CHUNK), counts, fm1, fm2, nchunk)


_EID, _COUNTS, _FM1, _FM2, NCHUNK = _build_tables()


def _rsqrt_newton(d):
    i = plsc.bitcast(d, jnp.int32)
    i = jnp.int32(0x5F3759DF) - lax.shift_right_arithmetic(i, jnp.int32(1))
    y = plsc.bitcast(i, jnp.float32)
    half_d = d * jnp.float32(0.5)
    for _ in range(3):
        y = y * (jnp.float32(1.5) - half_d * y * y)
    return y


def _sc1_body(x_ref, src_ref, dst_ref, eid_ref, cnt_ref,
              deg_ref, u_ref, srcact_ref, dstact_ref,
              eid_v, src_f, dgat0, dgat1, dstact, cnt_v, ones_v, degbuf,
              dinv_v, xbuf,
              deg_sp, sem_a, sem_b, sem_s):
    c = lax.axis_index("c")
    s = lax.axis_index("s")
    tile_r0 = s * ROWS_PER_TILE
    view_r0 = c * NPAD

    pltpu.sync_copy(cnt_ref.at[c], cnt_v)
    pltpu.sync_copy(eid_ref.at[c, s], eid_v)
    lanes = lax.iota(jnp.int32, 16)
    cnt = jnp.sum(jnp.where(lanes == s, cnt_v[...], jnp.int32(0)))
    nj = lax.div(cnt + jnp.int32(CHUNK - 1), jnp.int32(CHUNK))

    def fire_dst(j, buf, sem):
        eid_j = eid_v.at[pl.ds(j * CHUNK, CHUNK)]
        pltpu.async_copy(dst_ref.at[eid_j], buf, sem)

    def wait_dst(buf, sem):
        pltpu.make_async_copy(dst_ref.at[pl.ds(0, CHUNK)], buf, sem).wait()

    fire_dst(0, dgat0, sem_a)
    pltpu.async_copy(src_ref.at[eid_v], src_f, sem_b).wait()
    for k in range(CHUNK // 16):
        degbuf[pl.ds(k * 16, 16)] = jnp.zeros((16,), jnp.float32)
        ones_v[pl.ds(k * 16, 16)] = jnp.ones((16,), jnp.float32)
    for q in range(ROWS_PER_TILE // CHUNK):
        pltpu.sync_copy(degbuf, deg_sp.at[pl.ds(tile_r0 + q * CHUNK, CHUNK)])
    plsc.subcore_barrier()

    def process(j, buf):
        for k in range(CHUNK // 16):
            sv = src_f[pl.ds(j * CHUNK + k * 16, 16)]
            dv = buf[pl.ds(k * 16, 16)]
            pos = j * CHUNK + k * 16 + lanes
            live = pos < cnt
            dump = jnp.int32(N) + (pos & jnp.int32(127))
            spread = (pos * jnp.int32(37)) & jnp.int32(8191)
            src_f[pl.ds(j * CHUNK + k * 16, 16)] = (
                jnp.where(live, sv, spread) + view_r0)
            dstact[j, pl.ds(k * 16, 16)] = jnp.where(live, dv, dump)
        pltpu.async_copy(ones_v, deg_sp.at[dstact.at[j]], sem_s, add=True)

    @pl.loop(0, nj)
    def _phase_a(j):
        even = (j & jnp.int32(1)) == jnp.int32(0)

        @pl.when(even)
        def _():
            wait_dst(dgat0, sem_a)

            @pl.when(j + 1 < nj)
            def _():
                fire_dst(j + 1, dgat1, sem_b)

            process(j, dgat0)

        @pl.when(jnp.logical_not(even))
        def _():
            wait_dst(dgat1, sem_b)

            @pl.when(j + 1 < nj)
            def _():
                fire_dst(j + 1, dgat0, sem_a)

            process(j, dgat1)

    @pl.loop(0, nj)
    def _drain(j):
        pltpu.make_async_copy(dst_ref.at[pl.ds(0, CHUNK)], ones_v,
                              sem_s).wait()

    plsc.subcore_barrier()

    for q in range(ROWS_PER_TILE // CHUNK):
        r0 = tile_r0 + q * CHUNK
        pltpu.sync_copy(deg_sp.at[pl.ds(r0, CHUNK)], degbuf)
        for k in range(CHUNK // 16):
            d = degbuf[pl.ds(k * 16, 16)] + jnp.float32(1.0)
            dinv_v[pl.ds(q * CHUNK + k * 16, 16)] = _rsqrt_newton(d)
        pltpu.sync_copy(x_ref.at[pl.ds(r0, CHUNK)], xbuf)

        @pl.loop(0, CHUNK // 16)
        def _scale_rows(g):
            dv = dinv_v[pl.ds(q * CHUNK + g * 16, 16)]
            for r in range(16):
                av = jnp.full((16,), dv[r], jnp.float32)
                row = g * 16 + r
                for k in range(D // 16):
                    xbuf[row, pl.ds(k * 16, 16)] = (
                        xbuf[row, pl.ds(k * 16, 16)] * av)

        pltpu.sync_copy(xbuf, u_ref.at[pl.ds(view_r0 + r0, CHUNK)])

    pltpu.sync_copy(src_f, srcact_ref.at[c, s])
    pltpu.sync_copy(dstact, dstact_ref.at[c, s])
    pltpu.sync_copy(deg_sp.at[pl.ds(tile_r0, ROWS_PER_TILE)],
                    deg_ref.at[pl.ds(view_r0 + tile_r0, ROWS_PER_TILE)])


def _sc1(x_pad, src, dst, eid, counts):
    mesh = plsc.VectorSubcoreMesh(core_axis_name="c", subcore_axis_name="s")
    kern = pl.kernel(
        _sc1_body,
        out_type=[
            jax.ShapeDtypeStruct((NC * NPAD,), jnp.float32),
            jax.ShapeDtypeStruct((NC * NPAD, D), jnp.float32),
            jax.ShapeDtypeStruct((NC, NS, NCHUNK * CHUNK), jnp.int32),
            jax.ShapeDtypeStruct((NC, NS, NCHUNK, CHUNK), jnp.int32),
        ],
        mesh=mesh,
        scratch_types=[
            pltpu.VMEM((NCHUNK * CHUNK,), jnp.int32),
            pltpu.VMEM((NCHUNK * CHUNK,), jnp.int32),
            pltpu.VMEM((CHUNK,), jnp.int32),
            pltpu.VMEM((CHUNK,), jnp.int32),
            pltpu.VMEM((NCHUNK, CHUNK), jnp.int32),
            pltpu.VMEM((16,), jnp.int32),
            pltpu.VMEM((CHUNK,), jnp.float32),
            pltpu.VMEM((CHUNK,), jnp.float32),
            pltpu.VMEM((ROWS_PER_TILE,), jnp.float32),
            pltpu.VMEM((CHUNK, D), jnp.float32),
            pltpu.MemorySpace.VMEM_SHARED((NPAD,), jnp.float32),
            pltpu.SemaphoreType.DMA,
            pltpu.SemaphoreType.DMA,
            pltpu.SemaphoreType.DMA,
        ],
        compiler_params=pltpu.CompilerParams(needs_layout_passes=False),
    )
    return kern(x_pad, src, dst, eid, counts)


def _sc2_body(u_ref, srcact_ref, dstact_ref, cnt_ref,
              agg_ref,
              src_f, dstact, cnt_v, rowbuf0, rowbuf1,
              agg_sp, sem_a, sem_b, sem_s):
    c = lax.axis_index("c")
    s = lax.axis_index("s")
    tile_r0 = s * ROWS_PER_TILE
    view_r0 = c * NPAD

    pltpu.sync_copy(cnt_ref.at[c], cnt_v)
    pltpu.sync_copy(srcact_ref.at[c, s], src_f)
    pltpu.sync_copy(dstact_ref.at[c, s], dstact)
    lanes = lax.iota(jnp.int32, 16)
    cnt = jnp.sum(jnp.where(lanes == s, cnt_v[...], jnp.int32(0)))
    nj = lax.div(cnt + jnp.int32(CHUNK - 1), jnp.int32(CHUNK))

    def fire_gather(j, buf, sem):
        idx = src_f.at[pl.ds(j * CHUNK, CHUNK)]
        pltpu.async_copy(u_ref.at[idx], buf, sem)

    def wait_rows(buf, sem):
        pltpu.make_async_copy(u_ref.at[pl.ds(0, CHUNK)], buf, sem).wait()

    fire_gather(0, rowbuf0, sem_a)

    @pl.loop(0, CHUNK)
    def _zero_rb1(r):
        for k in range(D // 16):
            rowbuf1[r, pl.ds(k * 16, 16)] = jnp.zeros((16,), jnp.float32)

    for q in range(ROWS_PER_TILE // CHUNK):
        pltpu.sync_copy(rowbuf1, agg_sp.at[pl.ds(tile_r0 + q * CHUNK, CHUNK)])
    plsc.subcore_barrier()

    @pl.loop(0, nj)
    def _phase_c(j):
        even = (j & jnp.int32(1)) == jnp.int32(0)

        @pl.when(j > 0)
        def _():
            wait_rows(rowbuf0, sem_s)

        @pl.when(even)
        def _():
            wait_rows(rowbuf0, sem_a)

            @pl.when(j + 1 < nj)
            def _():
                fire_gather(j + 1, rowbuf1, sem_b)

            pltpu.async_copy(rowbuf0, agg_sp.at[dstact.at[j]], sem_s,
                             add=True)

        @pl.when(jnp.logical_not(even))
        def _():
            wait_rows(rowbuf1, sem_b)

            @pl.when(j + 1 < nj)
            def _():
                fire_gather(j + 1, rowbuf0, sem_a)

            pltpu.async_copy(rowbuf1, agg_sp.at[dstact.at[j]], sem_s,
                             add=True)

    wait_rows(rowbuf0, sem_s)

    plsc.subcore_barrier()

    pltpu.sync_copy(agg_sp.at[pl.ds(tile_r0, ROWS_PER_TILE)],
                    agg_ref.at[pl.ds(view_r0 + tile_r0, ROWS_PER_TILE)])


def _sc2(u, srcact, dstact, counts):
    mesh = plsc.VectorSubcoreMesh(core_axis_name="c", subcore_axis_name="s")
    kern = pl.kernel(
        _sc2_body,
        out_type=[
            jax.ShapeDtypeStruct((NC * NPAD, D), jnp.float32),
        ],
        mesh=mesh,
        scratch_types=[
            pltpu.VMEM((NCHUNK * CHUNK,), jnp.int32),
            pltpu.VMEM((NCHUNK, CHUNK), jnp.int32),
            pltpu.VMEM((16,), jnp.int32),
            pltpu.VMEM((CHUNK, D), jnp.float32),
            pltpu.VMEM((CHUNK, D), jnp.float32),
            pltpu.MemorySpace.VMEM_SHARED((NPAD, D), jnp.float32),
            pltpu.SemaphoreType.DMA,
            pltpu.SemaphoreType.DMA,
            pltpu.SemaphoreType.DMA,
        ],
        compiler_params=pltpu.CompilerParams(needs_layout_passes=False),
    )
    (agg,) = kern(u, srcact, dstact, counts)
    return agg


def _tc2_body(alpha_ref,
              a1_ref, u1_ref, d1_ref, a2_ref, u2_ref, d2_ref,
              w1_ref, w2_ref, bg_ref, wd1_ref, bd1_ref, wd2_ref, bd2_ref,
              o1_ref, o2_ref):
    alpha = alpha_ref[0]
    dot = functools.partial(
        lax.dot_general,
        dimension_numbers=(((1,), (0,)), ((), ())),
        precision=lax.Precision.HIGHEST,
        preferred_element_type=jnp.float32,
    )
    for a_ref, uu_ref, dd_ref, w_ref, o_ref in (
            (a1_ref, u1_ref, d1_ref, w1_ref, o1_ref),
            (a2_ref, u2_ref, d2_ref, w2_ref, o2_ref)):
        dinv = lax.rsqrt(dd_ref[...] + jnp.float32(1.0))
        zin = (a_ref[...] + uu_ref[...]) * dinv
        z = dot(zin, w_ref[...]) + bg_ref[...]
        z = jnp.where(z > 0, z, alpha * z)
        e = dot(z, wd1_ref[...]) + bd1_ref[...]
        e = jnp.where(e > 0, e, jnp.exp(e) - jnp.float32(1.0))
        o_ref[...] = dot(e, wd2_ref[...]) + bd2_ref[...]


def _tc2(agg, u, deg2, w1e, w2e, b_gcn, alpha, w_d1, b_d1, w_d2, b_d2):
    blk = 512
    nb = NPAD // blk
    rows1 = pl.BlockSpec((blk, D), lambda i: (i, 0))
    rows2 = pl.BlockSpec((blk, D), lambda i: (nb + i, 0))
    dcol1 = pl.BlockSpec((blk, 1), lambda i: (i, 0))
    dcol2 = pl.BlockSpec((blk, 1), lambda i: (nb + i, 0))
    full = pl.BlockSpec((D, D), lambda i: (0, 0))
    vec = pl.BlockSpec((1, D), lambda i: (0, 0))
    return pl.pallas_call(
        _tc2_body,
        grid=(nb,),
        in_specs=[
            pl.BlockSpec(memory_space=pltpu.SMEM),
            rows1, rows1, dcol1, rows2, rows2, dcol2,
            full, full, vec, full, vec, full, vec,
        ],
        out_specs=[rows1, rows1],
        out_shape=[
            jax.ShapeDtypeStruct((NPAD, D), jnp.float32),
            jax.ShapeDtypeStruct((NPAD, D), jnp.float32),
        ],
    )(alpha.reshape(1), agg, u, deg2, agg, u, deg2,
      w1e, w2e, b_gcn.reshape(1, D),
      w_d1, b_d1.reshape(1, D), w_d2, b_d2.reshape(1, D))


def kernel(x, edge_index, W_gcn, b_gcn, alpha, W_d1, b_d1, W_d2, b_d2):
    x_pad = jnp.pad(x, ((0, NPAD - N), (0, 0)))
    src = edge_index[0].astype(jnp.int32)
    dst = edge_index[1].astype(jnp.int32)
    eid = jnp.asarray(_EID)
    counts = jnp.asarray(_COUNTS)

    deg2, u, srcact, dstact = _sc1(x_pad, src, dst, eid, counts)
    agg = _sc2(u, srcact, dstact, counts)

    w1e = W_gcn * jnp.asarray(_FM1).reshape(D, 1)
    w2e = W_gcn * jnp.asarray(_FM2).reshape(D, 1)

    h1, h2 = _tc2(agg, u, deg2.reshape(NC * NPAD, 1), w1e, w2e, b_gcn, alpha,
                  W_d1, b_d1, W_d2, b_d2)
    return (h1[:N], h2[:N])


# speedup vs baseline: 58.3971x; 1.1722x over previous
import functools
import math

import jax
import jax.numpy as jnp
import numpy as np
from jax import lax
from jax.experimental import pallas as pl
from jax.experimental.pallas import tpu as pltpu
from jax.experimental.pallas import tpu_sc as plsc

N = 10000
E = 320000
D = 128
NS = 16
NC = 2
WIN = E // NS
NPAD = 10240
ROWS_PER_TILE = NPAD // NS
CHUNK = 128


def _tf2x32(k1, k2, x0, x1):
    rot = [np.array([13, 15, 26, 6], np.uint32),
           np.array([17, 29, 16, 24], np.uint32)]

    def rl(x, d):
        return ((x << d) | (x >> np.uint32(32 - d))).astype(np.uint32)

    ks = [k1, k2, (k1 ^ k2 ^ np.uint32(0x1BD11BDA)).astype(np.uint32)]
    x = [(x0 + ks[0]).astype(np.uint32), (x1 + ks[1]).astype(np.uint32)]
    for i in range(5):
        for r in rot[i % 2]:
            x[0] = (x[0] + x[1]).astype(np.uint32)
            x[1] = x[0] ^ rl(x[1], r)
        x[0] = (x[0] + ks[(i + 1) % 3]).astype(np.uint32)
        x[1] = (x[1] + ks[(i + 2) % 3] + np.uint32(i + 1)).astype(np.uint32)
    return x


def _np_split(key, num):
    hi = np.zeros(num, np.uint32)
    lo = np.arange(num, dtype=np.uint32)
    b1, b2 = _tf2x32(key[0], key[1], hi, lo)
    return np.stack([b1, b2], axis=1)


def _np_uniform(key, n):
    hi = np.zeros(n, np.uint32)
    lo = np.arange(n, dtype=np.uint32)
    b1, b2 = _tf2x32(key[0], key[1], hi, lo)
    fb = ((b1 ^ b2) >> np.uint32(9)) | np.uint32(0x3F800000)
    return fb.view(np.float32) - np.float32(1.0)


def _build_static_masks():
    ks = _np_split(np.array([0, 42], np.uint32), 4)
    keep1 = _np_uniform(ks[0], E) >= np.float32(0.8)
    keep2 = _np_uniform(ks[1], E) >= np.float32(0.7)
    fm1 = (_np_uniform(ks[2], D) >= np.float32(0.4)).astype(np.float32)
    fm2 = (_np_uniform(ks[3], D) >= np.float32(0.3)).astype(np.float32)
    return keep1, keep2, fm1, fm2


def _build_tables():
    keep1, keep2, fm1, fm2 = _build_static_masks()
    counts = np.zeros((NC, NS), np.int32)
    offs = [[None] * NS for _ in range(NC)]
    for v, keep in enumerate((keep1, keep2)):
        for t in range(NS):
            o = (np.nonzero(keep[t * WIN:(t + 1) * WIN])[0] + t * WIN).astype(
                np.int32)
            counts[v, t] = o.size
            offs[v][t] = o
    nchunk = int(math.ceil(counts.max() / CHUNK))
    eid = np.tile(np.arange(nchunk * CHUNK, dtype=np.int32) * 51 % WIN,
                  (NC, NS, 1))
    for v in range(NC):
        for t in range(NS):
            eid[v, t, : counts[v, t]] = offs[v][t] - t * WIN
    return (eid.reshape(NC, NS, nchunk * CHUNK), counts, fm1, fm2, nchunk)


_EID, _COUNTS, _FM1, _FM2, NCHUNK = _build_tables()


def _rsqrt_newton(d):
    i = plsc.bitcast(d, jnp.int32)
    i = jnp.int32(0x5F3759DF) - lax.shift_right_arithmetic(i, jnp.int32(1))
    y = plsc.bitcast(i, jnp.float32)
    half_d = d * jnp.float32(0.5)
    for _ in range(3):
        y = y * (jnp.float32(1.5) - half_d * y * y)
    return y


def _sc1_body(x_ref, src_ref, dst_ref, eid_ref, cnt_ref,
              deg_ref, u_ref, srcact_ref, dstact_ref,
              eid_v, src_f, src_win, dst_win, dstact, cnt_v, ones_v, degbuf,
              dinv_v, xbuf,
              deg_sp, sem_s):
    c = lax.axis_index("c")
    s = lax.axis_index("s")
    tile_r0 = s * ROWS_PER_TILE
    view_r0 = c * NPAD

    pltpu.sync_copy(cnt_ref.at[c], cnt_v)
    pltpu.sync_copy(eid_ref.at[c, s], eid_v)
    pltpu.sync_copy(src_ref.at[pl.ds(s * WIN, WIN)], src_win)
    pltpu.sync_copy(dst_ref.at[pl.ds(s * WIN, WIN)], dst_win)
    lanes = lax.iota(jnp.int32, 16)
    cnt = jnp.sum(jnp.where(lanes == s, cnt_v[...], jnp.int32(0)))
    nj = lax.div(cnt + jnp.int32(CHUNK - 1), jnp.int32(CHUNK))

    for k in range(CHUNK // 16):
        degbuf[pl.ds(k * 16, 16)] = jnp.zeros((16,), jnp.float32)
        ones_v[pl.ds(k * 16, 16)] = jnp.ones((16,), jnp.float32)
    for q in range(ROWS_PER_TILE // CHUNK):
        pltpu.sync_copy(degbuf, deg_sp.at[pl.ds(tile_r0 + q * CHUNK, CHUNK)])
    plsc.subcore_barrier()

    @pl.loop(0, nj)
    def _phase_a(j):
        for k in range(CHUNK // 16):
            lv = eid_v[pl.ds(j * CHUNK + k * 16, 16)]
            sv = plsc.load_gather(src_win, [lv])
            dv = plsc.load_gather(dst_win, [lv])
            pos = j * CHUNK + k * 16 + lanes
            live = pos < cnt
            dump = jnp.int32(N) + (pos & jnp.int32(127))
            spread = (pos * jnp.int32(37)) & jnp.int32(8191)
            src_f[pl.ds(j * CHUNK + k * 16, 16)] = (
                jnp.where(live, sv, spread) + view_r0)
            dstact[j, pl.ds(k * 16, 16)] = jnp.where(live, dv, dump)
        pltpu.async_copy(ones_v, deg_sp.at[dstact.at[j]], sem_s, add=True)

    @pl.loop(0, nj)
    def _drain(j):
        pltpu.make_async_copy(dst_ref.at[pl.ds(0, CHUNK)], ones_v,
                              sem_s).wait()

    plsc.subcore_barrier()

    for q in range(ROWS_PER_TILE // CHUNK):
        r0 = tile_r0 + q * CHUNK
        pltpu.sync_copy(deg_sp.at[pl.ds(r0, CHUNK)], degbuf)
        for k in range(CHUNK // 16):
            d = degbuf[pl.ds(k * 16, 16)] + jnp.float32(1.0)
            dinv_v[pl.ds(q * CHUNK + k * 16, 16)] = _rsqrt_newton(d)
        pltpu.sync_copy(x_ref.at[pl.ds(r0, CHUNK)], xbuf)

        @pl.loop(0, CHUNK // 16)
        def _scale_rows(g):
            dv = dinv_v[pl.ds(q * CHUNK + g * 16, 16)]
            for r in range(16):
                av = jnp.full((16,), dv[r], jnp.float32)
                row = g * 16 + r
                for k in range(D // 16):
                    xbuf[row, pl.ds(k * 16, 16)] = (
                        xbuf[row, pl.ds(k * 16, 16)] * av)

        pltpu.sync_copy(xbuf, u_ref.at[pl.ds(view_r0 + r0, CHUNK)])

    pltpu.sync_copy(src_f, srcact_ref.at[c, s])
    pltpu.sync_copy(dstact, dstact_ref.at[c, s])
    pltpu.sync_copy(deg_sp.at[pl.ds(tile_r0, ROWS_PER_TILE)],
                    deg_ref.at[pl.ds(view_r0 + tile_r0, ROWS_PER_TILE)])


def _sc1(x_pad, src, dst, eid, counts):
    mesh = plsc.VectorSubcoreMesh(core_axis_name="c", subcore_axis_name="s")
    kern = pl.kernel(
        _sc1_body,
        out_type=[
            jax.ShapeDtypeStruct((NC * NPAD,), jnp.float32),
            jax.ShapeDtypeStruct((NC * NPAD, D), jnp.float32),
            jax.ShapeDtypeStruct((NC, NS, NCHUNK * CHUNK), jnp.int32),
            jax.ShapeDtypeStruct((NC, NS, NCHUNK, CHUNK), jnp.int32),
        ],
        mesh=mesh,
        scratch_types=[
            pltpu.VMEM((NCHUNK * CHUNK,), jnp.int32),
            pltpu.VMEM((NCHUNK * CHUNK,), jnp.int32),
            pltpu.VMEM((WIN,), jnp.int32),
            pltpu.VMEM((WIN,), jnp.int32),
            pltpu.VMEM((NCHUNK, CHUNK), jnp.int32),
            pltpu.VMEM((16,), jnp.int32),
            pltpu.VMEM((CHUNK,), jnp.float32),
            pltpu.VMEM((CHUNK,), jnp.float32),
            pltpu.VMEM((ROWS_PER_TILE,), jnp.float32),
            pltpu.VMEM((CHUNK, D), jnp.float32),
            pltpu.MemorySpace.VMEM_SHARED((NPAD,), jnp.float32),
            pltpu.SemaphoreType.DMA,
        ],
        compiler_params=pltpu.CompilerParams(needs_layout_passes=False),
    )
    return kern(x_pad, src, dst, eid, counts)


def _sc2_body(u_ref, srcact_ref, dstact_ref, cnt_ref,
              agg_ref,
              src_f, dstact, cnt_v, rowbuf0, rowbuf1,
              agg_sp, sem_a, sem_b, sem_s):
    c = lax.axis_index("c")
    s = lax.axis_index("s")
    tile_r0 = s * ROWS_PER_TILE
    view_r0 = c * NPAD

    pltpu.sync_copy(cnt_ref.at[c], cnt_v)
    pltpu.sync_copy(srcact_ref.at[c, s], src_f)
    pltpu.sync_copy(dstact_ref.at[c, s], dstact)
    lanes = lax.iota(jnp.int32, 16)
    cnt = jnp.sum(jnp.where(lanes == s, cnt_v[...], jnp.int32(0)))
    nj = lax.div(cnt + jnp.int32(CHUNK - 1), jnp.int32(CHUNK))

    def fire_gather(j, buf, sem):
        idx = src_f.at[pl.ds(j * CHUNK, CHUNK)]
        pltpu.async_copy(u_ref.at[idx], buf, sem)

    def wait_rows(buf, sem):
        pltpu.make_async_copy(u_ref.at[pl.ds(0, CHUNK)], buf, sem).wait()

    fire_gather(0, rowbuf0, sem_a)

    @pl.loop(0, CHUNK)
    def _zero_rb1(r):
        for k in range(D // 16):
            rowbuf1[r, pl.ds(k * 16, 16)] = jnp.zeros((16,), jnp.float32)

    for q in range(ROWS_PER_TILE // CHUNK):
        pltpu.sync_copy(rowbuf1, agg_sp.at[pl.ds(tile_r0 + q * CHUNK, CHUNK)])
    plsc.subcore_barrier()

    @pl.loop(0, nj)
    def _phase_c(j):
        even = (j & jnp.int32(1)) == jnp.int32(0)

        @pl.when(j > 0)
        def _():
            wait_rows(rowbuf0, sem_s)

        @pl.when(even)
        def _():
            wait_rows(rowbuf0, sem_a)

            @pl.when(j + 1 < nj)
            def _():
                fire_gather(j + 1, rowbuf1, sem_b)

            pltpu.async_copy(rowbuf0, agg_sp.at[dstact.at[j]], sem_s,
                             add=True)

        @pl.when(jnp.logical_not(even))
        def _():
            wait_rows(rowbuf1, sem_b)

            @pl.when(j + 1 < nj)
            def _():
                fire_gather(j + 1, rowbuf0, sem_a)

            pltpu.async_copy(rowbuf1, agg_sp.at[dstact.at[j]], sem_s,
                             add=True)

    wait_rows(rowbuf0, sem_s)

    plsc.subcore_barrier()

    pltpu.sync_copy(agg_sp.at[pl.ds(tile_r0, ROWS_PER_TILE)],
                    agg_ref.at[pl.ds(view_r0 + tile_r0, ROWS_PER_TILE)])


def _sc2(u, srcact, dstact, counts):
    mesh = plsc.VectorSubcoreMesh(core_axis_name="c", subcore_axis_name="s")
    kern = pl.kernel(
        _sc2_body,
        out_type=[
            jax.ShapeDtypeStruct((NC * NPAD, D), jnp.float32),
        ],
        mesh=mesh,
        scratch_types=[
            pltpu.VMEM((NCHUNK * CHUNK,), jnp.int32),
            pltpu.VMEM((NCHUNK, CHUNK), jnp.int32),
            pltpu.VMEM((16,), jnp.int32),
            pltpu.VMEM((CHUNK, D), jnp.float32),
            pltpu.VMEM((CHUNK, D), jnp.float32),
            pltpu.MemorySpace.VMEM_SHARED((NPAD, D), jnp.float32),
            pltpu.SemaphoreType.DMA,
            pltpu.SemaphoreType.DMA,
            pltpu.SemaphoreType.DMA,
        ],
        compiler_params=pltpu.CompilerParams(needs_layout_passes=False),
    )
    (agg,) = kern(u, srcact, dstact, counts)
    return agg


def _tc2_body(alpha_ref,
              a1_ref, u1_ref, d1_ref, a2_ref, u2_ref, d2_ref,
              w1_ref, w2_ref, bg_ref, wd1_ref, bd1_ref, wd2_ref, bd2_ref,
              o1_ref, o2_ref):
    alpha = alpha_ref[0]
    dot = functools.partial(
        lax.dot_general,
        dimension_numbers=(((1,), (0,)), ((), ())),
        precision=lax.Precision.HIGHEST,
        preferred_element_type=jnp.float32,
    )
    for a_ref, uu_ref, dd_ref, w_ref, o_ref in (
            (a1_ref, u1_ref, d1_ref, w1_ref, o1_ref),
            (a2_ref, u2_ref, d2_ref, w2_ref, o2_ref)):
        dinv = lax.rsqrt(dd_ref[...] + jnp.float32(1.0))
        zin = (a_ref[...] + uu_ref[...]) * dinv
        z = dot(zin, w_ref[...]) + bg_ref[...]
        z = jnp.where(z > 0, z, alpha * z)
        e = dot(z, wd1_ref[...]) + bd1_ref[...]
        e = jnp.where(e > 0, e, jnp.exp(e) - jnp.float32(1.0))
        o_ref[...] = dot(e, wd2_ref[...]) + bd2_ref[...]


def _tc2(agg, u, deg2, w1e, w2e, b_gcn, alpha, w_d1, b_d1, w_d2, b_d2):
    blk = 512
    nb = NPAD // blk
    rows1 = pl.BlockSpec((blk, D), lambda i: (i, 0))
    rows2 = pl.BlockSpec((blk, D), lambda i: (nb + i, 0))
    dcol1 = pl.BlockSpec((blk, 1), lambda i: (i, 0))
    dcol2 = pl.BlockSpec((blk, 1), lambda i: (nb + i, 0))
    full = pl.BlockSpec((D, D), lambda i: (0, 0))
    vec = pl.BlockSpec((1, D), lambda i: (0, 0))
    return pl.pallas_call(
        _tc2_body,
        grid=(nb,),
        in_specs=[
            pl.BlockSpec(memory_space=pltpu.SMEM),
            rows1, rows1, dcol1, rows2, rows2, dcol2,
            full, full, vec, full, vec, full, vec,
        ],
        out_specs=[rows1, rows1],
        out_shape=[
            jax.ShapeDtypeStruct((NPAD, D), jnp.float32),
            jax.ShapeDtypeStruct((NPAD, D), jnp.float32),
        ],
    )(alpha.reshape(1), agg, u, deg2, agg, u, deg2,
      w1e, w2e, b_gcn.reshape(1, D),
      w_d1, b_d1.reshape(1, D), w_d2, b_d2.reshape(1, D))


def kernel(x, edge_index, W_gcn, b_gcn, alpha, W_d1, b_d1, W_d2, b_d2):
    x_pad = jnp.pad(x, ((0, NPAD - N), (0, 0)))
    src = edge_index[0].astype(jnp.int32)
    dst = edge_index[1].astype(jnp.int32)
    eid = jnp.asarray(_EID)
    counts = jnp.asarray(_COUNTS)

    deg2, u, srcact, dstact = _sc1(x_pad, src, dst, eid, counts)
    agg = _sc2(u, srcact, dstact, counts)

    w1e = W_gcn * jnp.asarray(_FM1).reshape(D, 1)
    w2e = W_gcn * jnp.asarray(_FM2).reshape(D, 1)

    h1, h2 = _tc2(agg, u, deg2.reshape(NC * NPAD, 1), w1e, w2e, b_gcn, alpha,
                  W_d1, b_d1, W_d2, b_d2)
    return (h1[:N], h2[:N])


# speedup vs baseline: 67.1056x; 1.1491x over previous
import functools
import math

import jax
import jax.numpy as jnp
import numpy as np
from jax import lax
from jax.experimental import pallas as pl
from jax.experimental.pallas import tpu as pltpu
from jax.experimental.pallas import tpu_sc as plsc

N = 10000
E = 320000
D = 128
NS = 16
NC = 2
WIN = E // NS
NPAD = 10240
ROWS_PER_TILE = NPAD // NS
CHUNK = 128


def _tf2x32(k1, k2, x0, x1):
    rot = [np.array([13, 15, 26, 6], np.uint32),
           np.array([17, 29, 16, 24], np.uint32)]

    def rl(x, d):
        return ((x << d) | (x >> np.uint32(32 - d))).astype(np.uint32)

    ks = [k1, k2, (k1 ^ k2 ^ np.uint32(0x1BD11BDA)).astype(np.uint32)]
    x = [(x0 + ks[0]).astype(np.uint32), (x1 + ks[1]).astype(np.uint32)]
    for i in range(5):
        for r in rot[i % 2]:
            x[0] = (x[0] + x[1]).astype(np.uint32)
            x[1] = x[0] ^ rl(x[1], r)
        x[0] = (x[0] + ks[(i + 1) % 3]).astype(np.uint32)
        x[1] = (x[1] + ks[(i + 2) % 3] + np.uint32(i + 1)).astype(np.uint32)
    return x


def _np_split(key, num):
    hi = np.zeros(num, np.uint32)
    lo = np.arange(num, dtype=np.uint32)
    b1, b2 = _tf2x32(key[0], key[1], hi, lo)
    return np.stack([b1, b2], axis=1)


def _np_uniform(key, n):
    hi = np.zeros(n, np.uint32)
    lo = np.arange(n, dtype=np.uint32)
    b1, b2 = _tf2x32(key[0], key[1], hi, lo)
    fb = ((b1 ^ b2) >> np.uint32(9)) | np.uint32(0x3F800000)
    return fb.view(np.float32) - np.float32(1.0)


def _build_static_masks():
    ks = _np_split(np.array([0, 42], np.uint32), 4)
    keep1 = _np_uniform(ks[0], E) >= np.float32(0.8)
    keep2 = _np_uniform(ks[1], E) >= np.float32(0.7)
    fm1 = (_np_uniform(ks[2], D) >= np.float32(0.4)).astype(np.float32)
    fm2 = (_np_uniform(ks[3], D) >= np.float32(0.3)).astype(np.float32)
    return keep1, keep2, fm1, fm2


def _build_tables():
    keep1, keep2, fm1, fm2 = _build_static_masks()
    counts = np.zeros((NC, NS), np.int32)
    offs = [[None] * NS for _ in range(NC)]
    for v, keep in enumerate((keep1, keep2)):
        for t in range(NS):
            o = (np.nonzero(keep[t * WIN:(t + 1) * WIN])[0] + t * WIN).astype(
                np.int32)
            counts[v, t] = o.size
            offs[v][t] = o
    nchunk = int(math.ceil(counts.max() / CHUNK))
    eid = np.tile(np.arange(nchunk * CHUNK, dtype=np.int32) * 51 % WIN,
                  (NC, NS, 1))
    for v in range(NC):
        for t in range(NS):
            eid[v, t, : counts[v, t]] = offs[v][t] - t * WIN
    return (eid.reshape(NC, NS, nchunk * CHUNK), counts, fm1, fm2, nchunk)


_EID, _COUNTS, _FM1, _FM2, NCHUNK = _build_tables()


def _rsqrt_newton(d):
    i = plsc.bitcast(d, jnp.int32)
    i = jnp.int32(0x5F3759DF) - lax.shift_right_arithmetic(i, jnp.int32(1))
    y = plsc.bitcast(i, jnp.float32)
    half_d = d * jnp.float32(0.5)
    for _ in range(3):
        y = y * (jnp.float32(1.5) - half_d * y * y)
    return y


def _sc1_body(x_ref, src_ref, dst_ref, eid_ref, cnt_ref,
              deg_ref, u_ref, srcact_ref, dstact_ref,
              eid_v, src_f, src_win, dst_win, dstact, cnt_v, ones_v, degbuf,
              dinv_v, xbuf,
              deg_sp, sem_s):
    c = lax.axis_index("c")
    s = lax.axis_index("s")
    tile_r0 = s * ROWS_PER_TILE
    view_r0 = c * NPAD

    pltpu.sync_copy(cnt_ref.at[c], cnt_v)
    pltpu.sync_copy(eid_ref.at[c, s], eid_v)
    pltpu.sync_copy(src_ref.at[pl.ds(s * WIN, WIN)], src_win)
    pltpu.sync_copy(dst_ref.at[pl.ds(s * WIN, WIN)], dst_win)
    lanes = lax.iota(jnp.int32, 16)
    cnt = jnp.sum(jnp.where(lanes == s, cnt_v[...], jnp.int32(0)))
    nj = lax.div(cnt + jnp.int32(CHUNK - 1), jnp.int32(CHUNK))

    for k in range(CHUNK // 16):
        degbuf[pl.ds(k * 16, 16)] = jnp.zeros((16,), jnp.float32)
        ones_v[pl.ds(k * 16, 16)] = jnp.ones((16,), jnp.float32)
    for q in range(ROWS_PER_TILE // CHUNK):
        pltpu.sync_copy(degbuf, deg_sp.at[pl.ds(tile_r0 + q * CHUNK, CHUNK)])
    plsc.subcore_barrier()

    @pl.loop(0, nj)
    def _phase_a(j):
        for k in range(CHUNK // 16):
            lv = eid_v[pl.ds(j * CHUNK + k * 16, 16)]
            sv = plsc.load_gather(src_win, [lv])
            dv = plsc.load_gather(dst_win, [lv])
            pos = j * CHUNK + k * 16 + lanes
            live = pos < cnt
            dump = jnp.int32(N) + (pos & jnp.int32(127))
            spread = (pos * jnp.int32(37)) & jnp.int32(8191)
            src_f[pl.ds(j * CHUNK + k * 16, 16)] = (
                jnp.where(live, sv, spread) + view_r0)
            dstact[j, pl.ds(k * 16, 16)] = jnp.where(live, dv, dump)
        pltpu.async_copy(ones_v, deg_sp.at[dstact.at[j]], sem_s, add=True)

    @pl.loop(0, nj)
    def _drain(j):
        pltpu.make_async_copy(dst_ref.at[pl.ds(0, CHUNK)], ones_v,
                              sem_s).wait()

    plsc.subcore_barrier()

    for q in range(ROWS_PER_TILE // CHUNK):
        r0 = tile_r0 + q * CHUNK
        pltpu.sync_copy(deg_sp.at[pl.ds(r0, CHUNK)], degbuf)
        for k in range(CHUNK // 16):
            d = degbuf[pl.ds(k * 16, 16)] + jnp.float32(1.0)
            dinv_v[pl.ds(q * CHUNK + k * 16, 16)] = _rsqrt_newton(d)
        pltpu.sync_copy(x_ref.at[pl.ds(r0, CHUNK)], xbuf)

        @pl.loop(0, CHUNK // 16)
        def _scale_rows(g):
            dv = dinv_v[pl.ds(q * CHUNK + g * 16, 16)]
            for r in range(16):
                av = jnp.full((16,), dv[r], jnp.float32)
                row = g * 16 + r
                for k in range(D // 16):
                    xbuf[row, pl.ds(k * 16, 16)] = (
                        xbuf[row, pl.ds(k * 16, 16)] * av)

        pltpu.sync_copy(xbuf, u_ref.at[pl.ds(view_r0 + r0, CHUNK)])

    pltpu.sync_copy(src_f, srcact_ref.at[c, s])
    pltpu.sync_copy(dstact, dstact_ref.at[c, s])
    pltpu.sync_copy(deg_sp.at[pl.ds(tile_r0, ROWS_PER_TILE)],
                    deg_ref.at[pl.ds(view_r0 + tile_r0, ROWS_PER_TILE)])


def _sc1(x_pad, src, dst, eid, counts):
    mesh = plsc.VectorSubcoreMesh(core_axis_name="c", subcore_axis_name="s")
    kern = pl.kernel(
        _sc1_body,
        out_type=[
            jax.ShapeDtypeStruct((NC * NPAD,), jnp.float32),
            jax.ShapeDtypeStruct((NC * NPAD, D), jnp.float32),
            jax.ShapeDtypeStruct((NC, NS, NCHUNK * CHUNK), jnp.int32),
            jax.ShapeDtypeStruct((NC, NS, NCHUNK, CHUNK), jnp.int32),
        ],
        mesh=mesh,
        scratch_types=[
            pltpu.VMEM((NCHUNK * CHUNK,), jnp.int32),
            pltpu.VMEM((NCHUNK * CHUNK,), jnp.int32),
            pltpu.VMEM((WIN,), jnp.int32),
            pltpu.VMEM((WIN,), jnp.int32),
            pltpu.VMEM((NCHUNK, CHUNK), jnp.int32),
            pltpu.VMEM((16,), jnp.int32),
            pltpu.VMEM((CHUNK,), jnp.float32),
            pltpu.VMEM((CHUNK,), jnp.float32),
            pltpu.VMEM((ROWS_PER_TILE,), jnp.float32),
            pltpu.VMEM((CHUNK, D), jnp.float32),
            pltpu.MemorySpace.VMEM_SHARED((NPAD,), jnp.float32),
            pltpu.SemaphoreType.DMA,
        ],
        compiler_params=pltpu.CompilerParams(needs_layout_passes=False),
    )
    return kern(x_pad, src, dst, eid, counts)


def _sc2_body(u_ref, srcact_ref, dstact_ref, cnt_ref,
              agg_ref,
              src_f, dstact, cnt_v, rowbuf0, rowbuf1,
              agg_sp, sem_a, sem_b, sem_s):
    c = lax.axis_index("c")
    s = lax.axis_index("s")
    tile_r0 = s * ROWS_PER_TILE
    view_r0 = c * NPAD

    pltpu.sync_copy(cnt_ref.at[c], cnt_v)
    pltpu.sync_copy(srcact_ref.at[c, s], src_f)
    pltpu.sync_copy(dstact_ref.at[c, s], dstact)
    lanes = lax.iota(jnp.int32, 16)
    cnt = jnp.sum(jnp.where(lanes == s, cnt_v[...], jnp.int32(0)))
    nj = lax.div(cnt + jnp.int32(CHUNK - 1), jnp.int32(CHUNK))

    def fire_gather(j, buf, sem):
        idx = src_f.at[pl.ds(j * CHUNK, CHUNK)]
        pltpu.async_copy(u_ref.at[idx], buf, sem)

    def wait_rows(buf, sem):
        pltpu.make_async_copy(u_ref.at[pl.ds(0, CHUNK)], buf, sem).wait()

    fire_gather(0, rowbuf0, sem_a)

    @pl.loop(0, CHUNK)
    def _zero_rb1(r):
        for k in range(D // 16):
            rowbuf1[r, pl.ds(k * 16, 16)] = jnp.zeros((16,), jnp.float32)

    for q in range(ROWS_PER_TILE // CHUNK):
        pltpu.sync_copy(rowbuf1, agg_sp.at[pl.ds(tile_r0 + q * CHUNK, CHUNK)])
    plsc.subcore_barrier()

    @pl.loop(0, nj)
    def _phase_c(j):
        even = (j & jnp.int32(1)) == jnp.int32(0)

        @pl.when(j > 0)
        def _():
            wait_rows(rowbuf0, sem_s)

        @pl.when(even)
        def _():
            wait_rows(rowbuf0, sem_a)

            @pl.when(j + 1 < nj)
            def _():
                fire_gather(j + 1, rowbuf1, sem_b)

            pltpu.async_copy(rowbuf0, agg_sp.at[dstact.at[j]], sem_s,
                             add=True)

        @pl.when(jnp.logical_not(even))
        def _():
            wait_rows(rowbuf1, sem_b)

            @pl.when(j + 1 < nj)
            def _():
                fire_gather(j + 1, rowbuf0, sem_a)

            pltpu.async_copy(rowbuf1, agg_sp.at[dstact.at[j]], sem_s,
                             add=True)

    wait_rows(rowbuf0, sem_s)

    plsc.subcore_barrier()

    pltpu.sync_copy(agg_sp.at[pl.ds(tile_r0, ROWS_PER_TILE)],
                    agg_ref.at[pl.ds(view_r0 + tile_r0, ROWS_PER_TILE)])


def _sc2(u, srcact, dstact, counts):
    mesh = plsc.VectorSubcoreMesh(core_axis_name="c", subcore_axis_name="s")
    kern = pl.kernel(
        _sc2_body,
        out_type=[
            jax.ShapeDtypeStruct((NC * NPAD, D), jnp.float32),
        ],
        mesh=mesh,
        scratch_types=[
            pltpu.VMEM((NCHUNK * CHUNK,), jnp.int32),
            pltpu.VMEM((NCHUNK, CHUNK), jnp.int32),
            pltpu.VMEM((16,), jnp.int32),
            pltpu.VMEM((CHUNK, D), jnp.float32),
            pltpu.VMEM((CHUNK, D), jnp.float32),
            pltpu.MemorySpace.VMEM_SHARED((NPAD, D), jnp.float32),
            pltpu.SemaphoreType.DMA,
            pltpu.SemaphoreType.DMA,
            pltpu.SemaphoreType.DMA,
        ],
        compiler_params=pltpu.CompilerParams(needs_layout_passes=False),
    )
    (agg,) = kern(u, srcact, dstact, counts)
    return agg


def _tc2_body(alpha_ref,
              a1_ref, u1_ref, d1_ref, a2_ref, u2_ref, d2_ref,
              w1_ref, w2_ref, bg_ref, wd1_ref, bd1_ref, wd2_ref, bd2_ref,
              o1_ref, o2_ref):
    alpha = alpha_ref[0]
    dot = functools.partial(
        lax.dot_general,
        dimension_numbers=(((1,), (0,)), ((), ())),
        precision=lax.Precision.DEFAULT,
        preferred_element_type=jnp.float32,
    )
    for a_ref, uu_ref, dd_ref, w_ref, o_ref in (
            (a1_ref, u1_ref, d1_ref, w1_ref, o1_ref),
            (a2_ref, u2_ref, d2_ref, w2_ref, o2_ref)):
        dinv = lax.rsqrt(dd_ref[...] + jnp.float32(1.0))
        zin = (a_ref[...] + uu_ref[...]) * dinv
        z = dot(zin, w_ref[...]) + bg_ref[...]
        z = jnp.where(z > 0, z, alpha * z)
        e = dot(z, wd1_ref[...]) + bd1_ref[...]
        e = jnp.where(e > 0, e, jnp.exp(e) - jnp.float32(1.0))
        o_ref[...] = dot(e, wd2_ref[...]) + bd2_ref[...]


def _tc2(agg, u, deg2, w1e, w2e, b_gcn, alpha, w_d1, b_d1, w_d2, b_d2):
    blk = 512
    nb = NPAD // blk
    rows1 = pl.BlockSpec((blk, D), lambda i: (i, 0))
    rows2 = pl.BlockSpec((blk, D), lambda i: (nb + i, 0))
    dcol1 = pl.BlockSpec((blk, 1), lambda i: (i, 0))
    dcol2 = pl.BlockSpec((blk, 1), lambda i: (nb + i, 0))
    full = pl.BlockSpec((D, D), lambda i: (0, 0))
    vec = pl.BlockSpec((1, D), lambda i: (0, 0))
    return pl.pallas_call(
        _tc2_body,
        grid=(nb,),
        in_specs=[
            pl.BlockSpec(memory_space=pltpu.SMEM),
            rows1, rows1, dcol1, rows2, rows2, dcol2,
            full, full, vec, full, vec, full, vec,
        ],
        out_specs=[rows1, rows1],
        out_shape=[
            jax.ShapeDtypeStruct((NPAD, D), jnp.float32),
            jax.ShapeDtypeStruct((NPAD, D), jnp.float32),
        ],
    )(alpha.reshape(1), agg, u, deg2, agg, u, deg2,
      w1e, w2e, b_gcn.reshape(1, D),
      w_d1, b_d1.reshape(1, D), w_d2, b_d2.reshape(1, D))


def kernel(x, edge_index, W_gcn, b_gcn, alpha, W_d1, b_d1, W_d2, b_d2):
    x_pad = jnp.pad(x, ((0, NPAD - N), (0, 0)))
    src = edge_index[0].astype(jnp.int32)
    dst = edge_index[1].astype(jnp.int32)
    eid = jnp.asarray(_EID)
    counts = jnp.asarray(_COUNTS)

    deg2, u, srcact, dstact = _sc1(x_pad, src, dst, eid, counts)
    agg = _sc2(u, srcact, dstact, counts)

    w1e = W_gcn * jnp.asarray(_FM1).reshape(D, 1)
    w2e = W_gcn * jnp.asarray(_FM2).reshape(D, 1)

    h1, h2 = _tc2(agg, u, deg2.reshape(NC * NPAD, 1), w1e, w2e, b_gcn, alpha,
                  W_d1, b_d1, W_d2, b_d2)
    return (h1[:N], h2[:N])


# speedup vs baseline: 70.2017x; 1.0461x over previous
import functools
import math

import jax
import jax.numpy as jnp
import numpy as np
from jax import lax
from jax.experimental import pallas as pl
from jax.experimental.pallas import tpu as pltpu
from jax.experimental.pallas import tpu_sc as plsc

N = 10000
E = 320000
D = 128
NS = 16
NC = 2
WIN = E // NS
NPAD = 10240
ROWS_PER_TILE = NPAD // NS
CHUNK = 128


def _tf2x32(k1, k2, x0, x1):
    rot = [np.array([13, 15, 26, 6], np.uint32),
           np.array([17, 29, 16, 24], np.uint32)]

    def rl(x, d):
        return ((x << d) | (x >> np.uint32(32 - d))).astype(np.uint32)

    ks = [k1, k2, (k1 ^ k2 ^ np.uint32(0x1BD11BDA)).astype(np.uint32)]
    x = [(x0 + ks[0]).astype(np.uint32), (x1 + ks[1]).astype(np.uint32)]
    for i in range(5):
        for r in rot[i % 2]:
            x[0] = (x[0] + x[1]).astype(np.uint32)
            x[1] = x[0] ^ rl(x[1], r)
        x[0] = (x[0] + ks[(i + 1) % 3]).astype(np.uint32)
        x[1] = (x[1] + ks[(i + 2) % 3] + np.uint32(i + 1)).astype(np.uint32)
    return x


def _np_split(key, num):
    hi = np.zeros(num, np.uint32)
    lo = np.arange(num, dtype=np.uint32)
    b1, b2 = _tf2x32(key[0], key[1], hi, lo)
    return np.stack([b1, b2], axis=1)


def _np_uniform(key, n):
    hi = np.zeros(n, np.uint32)
    lo = np.arange(n, dtype=np.uint32)
    b1, b2 = _tf2x32(key[0], key[1], hi, lo)
    fb = ((b1 ^ b2) >> np.uint32(9)) | np.uint32(0x3F800000)
    return fb.view(np.float32) - np.float32(1.0)


def _build_static_masks():
    ks = _np_split(np.array([0, 42], np.uint32), 4)
    keep1 = _np_uniform(ks[0], E) >= np.float32(0.8)
    keep2 = _np_uniform(ks[1], E) >= np.float32(0.7)
    fm1 = (_np_uniform(ks[2], D) >= np.float32(0.4)).astype(np.float32)
    fm2 = (_np_uniform(ks[3], D) >= np.float32(0.3)).astype(np.float32)
    return keep1, keep2, fm1, fm2


def _build_tables():
    keep1, keep2, fm1, fm2 = _build_static_masks()
    counts = np.zeros((NC, NS), np.int32)
    offs = [[None] * NS for _ in range(NC)]
    for v, keep in enumerate((keep1, keep2)):
        for t in range(NS):
            o = (np.nonzero(keep[t * WIN:(t + 1) * WIN])[0] + t * WIN).astype(
                np.int32)
            counts[v, t] = o.size
            offs[v][t] = o
    nchunk = int(math.ceil(counts.max() / CHUNK))
    eid = np.tile(np.arange(nchunk * CHUNK, dtype=np.int32) * 51 % WIN,
                  (NC, NS, 1))
    for v in range(NC):
        for t in range(NS):
            eid[v, t, : counts[v, t]] = offs[v][t] - t * WIN
    return (eid.reshape(NC, NS, nchunk * CHUNK), counts, fm1, fm2, nchunk)


_EID, _COUNTS, _FM1, _FM2, NCHUNK = _build_tables()


def _rsqrt_newton(d):
    i = plsc.bitcast(d, jnp.int32)
    i = jnp.int32(0x5F3759DF) - lax.shift_right_arithmetic(i, jnp.int32(1))
    y = plsc.bitcast(i, jnp.float32)
    half_d = d * jnp.float32(0.5)
    for _ in range(3):
        y = y * (jnp.float32(1.5) - half_d * y * y)
    return y


def _sc1_body(x_ref, src_ref, dst_ref, eid_ref, cnt_ref,
              deg_ref, u_ref, srcact_ref, dstact_ref,
              eid_v, src_f, src_win, dst_win, dstact, cnt_v, ones_v, degbuf,
              dinv_v, xbuf,
              deg_sp, sem_s):
    c = lax.axis_index("c")
    s = lax.axis_index("s")
    tile_r0 = s * ROWS_PER_TILE
    view_r0 = c * NPAD

    pltpu.sync_copy(cnt_ref.at[c], cnt_v)
    pltpu.sync_copy(eid_ref.at[c, s], eid_v)
    pltpu.sync_copy(src_ref.at[pl.ds(s * WIN, WIN)], src_win)
    pltpu.sync_copy(dst_ref.at[pl.ds(s * WIN, WIN)], dst_win)
    lanes = lax.iota(jnp.int32, 16)
    cnt = jnp.sum(jnp.where(lanes == s, cnt_v[...], jnp.int32(0)))
    nj = lax.div(cnt + jnp.int32(CHUNK - 1), jnp.int32(CHUNK))

    for k in range(CHUNK // 16):
        degbuf[pl.ds(k * 16, 16)] = jnp.zeros((16,), jnp.float32)
        ones_v[pl.ds(k * 16, 16)] = jnp.ones((16,), jnp.float32)
    for q in range(ROWS_PER_TILE // CHUNK):
        pltpu.sync_copy(degbuf, deg_sp.at[pl.ds(tile_r0 + q * CHUNK, CHUNK)])
    plsc.subcore_barrier()

    @pl.loop(0, nj)
    def _phase_a(j):
        for k in range(CHUNK // 16):
            lv = eid_v[pl.ds(j * CHUNK + k * 16, 16)]
            sv = plsc.load_gather(src_win, [lv])
            dv = plsc.load_gather(dst_win, [lv])
            pos = j * CHUNK + k * 16 + lanes
            live = pos < cnt
            dump = jnp.int32(N) + (pos & jnp.int32(127))
            spread = (pos * jnp.int32(37)) & jnp.int32(8191)
            src_f[pl.ds(j * CHUNK + k * 16, 16)] = (
                jnp.where(live, sv, spread) + view_r0)
            dstact[j, pl.ds(k * 16, 16)] = jnp.where(live, dv, dump)
        pltpu.async_copy(ones_v, deg_sp.at[dstact.at[j]], sem_s, add=True)

    @pl.loop(0, nj)
    def _drain(j):
        pltpu.make_async_copy(dst_ref.at[pl.ds(0, CHUNK)], ones_v,
                              sem_s).wait()

    plsc.subcore_barrier()

    for q in range(ROWS_PER_TILE // CHUNK):
        r0 = tile_r0 + q * CHUNK
        pltpu.sync_copy(deg_sp.at[pl.ds(r0, CHUNK)], degbuf)
        for k in range(CHUNK // 16):
            d = degbuf[pl.ds(k * 16, 16)] + jnp.float32(1.0)
            dinv_v[pl.ds(q * CHUNK + k * 16, 16)] = _rsqrt_newton(d)
        pltpu.sync_copy(x_ref.at[pl.ds(r0, CHUNK)], xbuf)

        @pl.loop(0, CHUNK // 16)
        def _scale_rows(g):
            dv = dinv_v[pl.ds(q * CHUNK + g * 16, 16)]
            for r in range(16):
                av = jnp.full((16,), dv[r], jnp.float32)
                row = g * 16 + r
                for k in range(D // 16):
                    xbuf[row, pl.ds(k * 16, 16)] = (
                        xbuf[row, pl.ds(k * 16, 16)] * av)

        pltpu.sync_copy(xbuf, u_ref.at[pl.ds(view_r0 + r0, CHUNK)])

    pltpu.sync_copy(src_f, srcact_ref.at[c, s])
    pltpu.sync_copy(dstact, dstact_ref.at[c, s])
    pltpu.sync_copy(deg_sp.at[pl.ds(tile_r0, ROWS_PER_TILE)],
                    deg_ref.at[pl.ds(view_r0 + tile_r0, ROWS_PER_TILE)])


def _sc1(x_pad, src, dst, eid, counts):
    mesh = plsc.VectorSubcoreMesh(core_axis_name="c", subcore_axis_name="s")
    kern = pl.kernel(
        _sc1_body,
        out_type=[
            jax.ShapeDtypeStruct((NC * NPAD,), jnp.float32),
            jax.ShapeDtypeStruct((NC * NPAD, D), jnp.float32),
            jax.ShapeDtypeStruct((NC, NS, NCHUNK * CHUNK), jnp.int32),
            jax.ShapeDtypeStruct((NC, NS, NCHUNK, CHUNK), jnp.int32),
        ],
        mesh=mesh,
        scratch_types=[
            pltpu.VMEM((NCHUNK * CHUNK,), jnp.int32),
            pltpu.VMEM((NCHUNK * CHUNK,), jnp.int32),
            pltpu.VMEM((WIN,), jnp.int32),
            pltpu.VMEM((WIN,), jnp.int32),
            pltpu.VMEM((NCHUNK, CHUNK), jnp.int32),
            pltpu.VMEM((16,), jnp.int32),
            pltpu.VMEM((CHUNK,), jnp.float32),
            pltpu.VMEM((CHUNK,), jnp.float32),
            pltpu.VMEM((ROWS_PER_TILE,), jnp.float32),
            pltpu.VMEM((CHUNK, D), jnp.float32),
            pltpu.MemorySpace.VMEM_SHARED((NPAD,), jnp.float32),
            pltpu.SemaphoreType.DMA,
        ],
        compiler_params=pltpu.CompilerParams(needs_layout_passes=False),
    )
    return kern(x_pad, src, dst, eid, counts)


def _sc2_body(u_ref, srcact_ref, dstact_ref, cnt_ref,
              agg_ref,
              src_f, dstact, cnt_v, rowbuf0, rowbuf1,
              agg_sp, sem_a, sem_b, sem_s):
    c = lax.axis_index("c")
    s = lax.axis_index("s")
    tile_r0 = s * ROWS_PER_TILE
    view_r0 = c * NPAD

    pltpu.sync_copy(cnt_ref.at[c], cnt_v)
    pltpu.sync_copy(srcact_ref.at[c, s], src_f)
    pltpu.sync_copy(dstact_ref.at[c, s], dstact)
    lanes = lax.iota(jnp.int32, 16)
    cnt = jnp.sum(jnp.where(lanes == s, cnt_v[...], jnp.int32(0)))
    nj = lax.div(cnt + jnp.int32(CHUNK - 1), jnp.int32(CHUNK))

    def fire_gather(j, buf, sem):
        idx = src_f.at[pl.ds(j * CHUNK, CHUNK)]
        pltpu.async_copy(u_ref.at[idx], buf, sem)

    def wait_rows(buf, sem):
        pltpu.make_async_copy(u_ref.at[pl.ds(0, CHUNK)], buf, sem).wait()

    fire_gather(0, rowbuf0, sem_a)

    @pl.loop(0, CHUNK)
    def _zero_rb1(r):
        for k in range(D // 16):
            rowbuf1[r, pl.ds(k * 16, 16)] = jnp.zeros((16,), jnp.float32)

    for q in range(ROWS_PER_TILE // CHUNK):
        pltpu.sync_copy(rowbuf1, agg_sp.at[pl.ds(tile_r0 + q * CHUNK, CHUNK)])
    plsc.subcore_barrier()

    @pl.loop(0, nj)
    def _phase_c(j):
        even = (j & jnp.int32(1)) == jnp.int32(0)

        @pl.when(j > 0)
        def _():
            wait_rows(rowbuf0, sem_s)

        @pl.when(even)
        def _():
            wait_rows(rowbuf0, sem_a)

            @pl.when(j + 1 < nj)
            def _():
                fire_gather(j + 1, rowbuf1, sem_b)

            pltpu.async_copy(rowbuf0, agg_sp.at[dstact.at[j]], sem_s,
                             add=True)

        @pl.when(jnp.logical_not(even))
        def _():
            wait_rows(rowbuf1, sem_b)

            @pl.when(j + 1 < nj)
            def _():
                fire_gather(j + 1, rowbuf0, sem_a)

            pltpu.async_copy(rowbuf1, agg_sp.at[dstact.at[j]], sem_s,
                             add=True)

    wait_rows(rowbuf0, sem_s)

    plsc.subcore_barrier()

    pltpu.sync_copy(agg_sp.at[pl.ds(tile_r0, ROWS_PER_TILE)],
                    agg_ref.at[pl.ds(view_r0 + tile_r0, ROWS_PER_TILE)])


def _sc2(u, srcact, dstact, counts):
    mesh = plsc.VectorSubcoreMesh(core_axis_name="c", subcore_axis_name="s")
    kern = pl.kernel(
        _sc2_body,
        out_type=[
            jax.ShapeDtypeStruct((NC * NPAD, D), jnp.float32),
        ],
        mesh=mesh,
        scratch_types=[
            pltpu.VMEM((NCHUNK * CHUNK,), jnp.int32),
            pltpu.VMEM((NCHUNK, CHUNK), jnp.int32),
            pltpu.VMEM((16,), jnp.int32),
            pltpu.VMEM((CHUNK, D), jnp.float32),
            pltpu.VMEM((CHUNK, D), jnp.float32),
            pltpu.MemorySpace.VMEM_SHARED((NPAD, D), jnp.float32),
            pltpu.SemaphoreType.DMA,
            pltpu.SemaphoreType.DMA,
            pltpu.SemaphoreType.DMA,
        ],
        compiler_params=pltpu.CompilerParams(needs_layout_passes=False),
    )
    (agg,) = kern(u, srcact, dstact, counts)
    return agg


def _tc2_body(alpha_ref,
              a1_ref, u1_ref, d1_ref, a2_ref, u2_ref, d2_ref,
              w1_ref, w2_ref, bg_ref, wd1_ref, bd1_ref, wd2_ref, bd2_ref,
              o1_ref, o2_ref):
    alpha = alpha_ref[0]
    dot = functools.partial(
        lax.dot_general,
        dimension_numbers=(((1,), (0,)), ((), ())),
        precision=lax.Precision.DEFAULT,
        preferred_element_type=jnp.float32,
    )
    for a_ref, uu_ref, dd_ref, w_ref, o_ref in (
            (a1_ref, u1_ref, d1_ref, w1_ref, o1_ref),
            (a2_ref, u2_ref, d2_ref, w2_ref, o2_ref)):
        dinv = lax.rsqrt(dd_ref[...] + jnp.float32(1.0))
        zin = (a_ref[...] + uu_ref[...]) * dinv
        z = dot(zin, w_ref[...]) + bg_ref[...]
        z = jnp.where(z > 0, z, alpha * z)
        e = dot(z, wd1_ref[...]) + bd1_ref[...]
        e = jnp.where(e > 0, e, jnp.exp(e) - jnp.float32(1.0))
        o_ref[...] = dot(e, wd2_ref[...]) + bd2_ref[...]


def _tc2(agg, u, deg2, w1e, w2e, b_gcn, alpha, w_d1, b_d1, w_d2, b_d2):
    blk = 1024
    nb = NPAD // blk
    rows1 = pl.BlockSpec((blk, D), lambda i: (i, 0))
    rows2 = pl.BlockSpec((blk, D), lambda i: (nb + i, 0))
    dcol1 = pl.BlockSpec((blk, 1), lambda i: (i, 0))
    dcol2 = pl.BlockSpec((blk, 1), lambda i: (nb + i, 0))
    full = pl.BlockSpec((D, D), lambda i: (0, 0))
    vec = pl.BlockSpec((1, D), lambda i: (0, 0))
    return pl.pallas_call(
        _tc2_body,
        grid=(nb,),
        in_specs=[
            pl.BlockSpec(memory_space=pltpu.SMEM),
            rows1, rows1, dcol1, rows2, rows2, dcol2,
            full, full, vec, full, vec, full, vec,
        ],
        out_specs=[rows1, rows1],
        out_shape=[
            jax.ShapeDtypeStruct((NPAD, D), jnp.float32),
            jax.ShapeDtypeStruct((NPAD, D), jnp.float32),
        ],
    )(alpha.reshape(1), agg, u, deg2, agg, u, deg2,
      w1e, w2e, b_gcn.reshape(1, D),
      w_d1, b_d1.reshape(1, D), w_d2, b_d2.reshape(1, D))


def kernel(x, edge_index, W_gcn, b_gcn, alpha, W_d1, b_d1, W_d2, b_d2):
    x_pad = jnp.pad(x, ((0, NPAD - N), (0, 0)))
    src = edge_index[0].astype(jnp.int32)
    dst = edge_index[1].astype(jnp.int32)
    eid = jnp.asarray(_EID)
    counts = jnp.asarray(_COUNTS)

    deg2, u, srcact, dstact = _sc1(x_pad, src, dst, eid, counts)
    agg = _sc2(u, srcact, dstact, counts)

    w1e = W_gcn * jnp.asarray(_FM1).reshape(D, 1)
    w2e = W_gcn * jnp.asarray(_FM2).reshape(D, 1)

    h1, h2 = _tc2(agg, u, deg2.reshape(NC * NPAD, 1), w1e, w2e, b_gcn, alpha,
                  W_d1, b_d1, W_d2, b_d2)
    return (h1[:N], h2[:N])
